# baseline (speedup 1.0000x reference)
import jax
import jax.numpy as jnp
from jax import lax
from jax.experimental import pallas as pl
from jax.experimental.pallas import tpu as pltpu
from jax.experimental.pallas import tpu_sc as plsc

VEC = 64
NWORDS = 1000000
NMAIN = 999936
NTAIL = NWORDS - NMAIN
B = 4096
K = 21
NP = B * K

NC, NS = 2, 16
DPC = VEC // NC
NBAND = DPC // 8
PPT = NP // NS
BPT = B // NS
CP = 128
NCH = PPT // CP

_TS = [(i * (NMAIN // 128)) // NS * 128 for i in range(NS)] + [NMAIN]
_SEGS = [(_TS[i], _TS[i + 1] - _TS[i]) for i in range(NS)]


def _body(doc_hbm, tn_hbm, dt_hbm, o_hbm, dtail_hbm, otail_hbm,
          p0_hbm, p1_hbm,
          doc_v, tn_v, bmap_v, dt_all, g_v, acc_v, tails_v, blk_v,
          row_s,
          sem_st, sem_g, sem_d):
    cid = lax.axis_index("c")
    sid = lax.axis_index("s")
    dbase = cid * DPC

    pltpu.sync_copy(doc_hbm.at[pl.ds(sid * BPT, BPT)], doc_v.at[pl.ds(0, BPT)])
    pltpu.sync_copy(tn_hbm.at[pl.ds(sid * PPT, PPT)], tn_v)

    pltpu.sync_copy(dtail_hbm.at[pl.ds(dbase * NTAIL, DPC * NTAIL)],
                    tails_v.at[0])
    pltpu.sync_copy(otail_hbm.at[pl.ds(dbase * NTAIL, DPC * NTAIL)],
                    tails_v.at[1])

    iota = jnp.arange(16, dtype=jnp.int32)

    @pl.loop(0, PPT // 16)
    def init_maps(g):
        p = sid * PPT + g * 16 + iota
        bmap_v[pl.ds(g * 16, 16)] = p // K - sid * BPT
        acc_v[pl.ds(g * 16, 16)] = jnp.zeros((16,), jnp.float32)

    def fire_stage(tab_hbm, tail_idx, d):
        for i, (st, ln) in enumerate(_SEGS):
            @pl.when(sid == i)
            def _():
                pltpu.async_copy(tab_hbm.at[d, pl.ds(st, ln)],
                                 row_s.at[pl.ds(st, ln)], sem_st)

        @pl.when(sid == 0)
        def _():
            pltpu.async_copy(
                tails_v.at[tail_idx, pl.ds((d - dbase) * NTAIL, NTAIL)],
                row_s.at[pl.ds(NMAIN, NTAIL)], sem_st)

    def wait_stage():
        for i, (st, ln) in enumerate(_SEGS):
            @pl.when(sid == i)
            def _():
                pltpu.make_async_copy(o_hbm.at[0, pl.ds(st, ln)],
                                      row_s.at[pl.ds(st, ln)],
                                      sem_st).wait()

        @pl.when(sid == 0)
        def _():
            pltpu.make_async_copy(tails_v.at[0, pl.ds(0, NTAIL)],
                                  row_s.at[pl.ds(NMAIN, NTAIL)],
                                  sem_st).wait()

    fire_stage(o_hbm, 1, dbase)

    s16 = lax.rem(iota, 8)
    half = iota // 8

    def doc_step(band, gi):
        dvec = doc_v[pl.ds(gi * 8, 16)]
        rt8 = pl.multiple_of((cid * NBAND + band) * 8, 8)
        for j in range(8):
            w = dvec[j]
            w0 = pl.multiple_of(
                jnp.minimum(w // 128 * 128, NMAIN - 128), 128)
            pltpu.async_copy(dt_hbm.at[pl.ds(rt8, 8), pl.ds(w0, 128)],
                             blk_v.at[j], sem_d)
        for j2 in range(4):
            for j in (2 * j2, 2 * j2 + 1):
                pltpu.make_async_copy(
                    dt_hbm.at[pl.ds(0, 8), pl.ds(0, 128)],
                    blk_v.at[j], sem_d).wait()
            wa, wb = dvec[2 * j2], dvec[2 * j2 + 1]
            wvec = jnp.where(iota < 8, jnp.full((16,), 0, jnp.int32) + wa,
                             jnp.full((16,), 0, jnp.int32) + wb)
            lane_m = wvec - jnp.minimum(wvec // 128 * 128, NMAIN - 128)
            jt = jnp.maximum(wvec - NMAIN, 0)
            jvec = 2 * j2 + half
            vals_m = plsc.load_gather(blk_v, [jvec, s16, lane_m])
            tloc = band * 8 + s16
            vals_t = plsc.load_gather(
                tails_v, [jnp.zeros((16,), jnp.int32), tloc * NTAIL + jt])
            vals = jnp.where(wvec >= NMAIN, vals_t, vals_m)
            plsc.store_scatter(dt_all, [tloc, gi * 8 + jvec], vals)

    @pl.loop(0, BPT // 8)
    def doc_band0(gi):
        doc_step(0, gi)

    @pl.loop(0, DPC)
    def pass2(t):
        d = dbase + t
        band = t // 8 + 1

        @pl.when(band < NBAND)
        def _():
            @pl.loop(0, 4)
            def steps(q):
                doc_step(band, lax.rem(t, 8) * 4 + q)

        wait_stage()
        plsc.subcore_barrier()

        @pl.loop(0, NCH, unroll=6)
        def fire_g(c):
            pltpu.async_copy(row_s.at[tn_v.at[pl.ds(c * CP, CP)]],
                             g_v.at[pl.ds(c * CP, CP)], sem_g)

        pltpu.make_async_copy(o_hbm.at[0, pl.ds(0, PPT)], g_v, sem_g).wait()
        plsc.subcore_barrier()

        @pl.when(t + 1 < DPC)
        def _():
            fire_stage(o_hbm, 1, d + 1)

        tsplat = jnp.full((16,), 0, jnp.int32) + t

        @pl.loop(0, PPT // 16, unroll=4)
        def accum(g):
            bv = bmap_v[pl.ds(g * 16, 16)]
            dv = plsc.load_gather(dt_all, [tsplat, bv])
            plsc.addupdate(acc_v.at[pl.ds(g * 16, 16)],
                           dv * g_v[pl.ds(g * 16, 16)])

    @pl.when(cid == 0)
    def _():
        pltpu.sync_copy(acc_v, p0_hbm.at[pl.ds(sid * PPT, PPT)])

    @pl.when(cid == 1)
    def _():
        pltpu.sync_copy(acc_v, p1_hbm.at[pl.ds(sid * PPT, PPT)])


def _combine_body(p0_hbm, p1_hbm, out_hbm, v0, v1, vo):
    wid = lax.axis_index("s") * NC + lax.axis_index("c")
    n = NP // (NC * NS)
    pltpu.sync_copy(p0_hbm.at[pl.ds(wid * n, n)], v0)
    pltpu.sync_copy(p1_hbm.at[pl.ds(wid * n, n)], v1)

    @pl.loop(0, n // 16)
    def add(g):
        s = pl.ds(g * 16, 16)
        vo[s] = v0[s] + v1[s]

    pltpu.sync_copy(vo, out_hbm.at[pl.ds(wid * n, n)])


@jax.jit
def _run(doc_ids, tn_flat, dt_tab, o_tab, dtail, otail):
    mesh = plsc.VectorSubcoreMesh(core_axis_name="c", subcore_axis_name="s")
    params = pltpu.CompilerParams(use_tc_tiling_on_sc=True,
                                  needs_layout_passes=False)
    f = pl.kernel(
        _body,
        out_type=(jax.ShapeDtypeStruct((NP,), jnp.float32),
                  jax.ShapeDtypeStruct((NP,), jnp.float32)),
        mesh=mesh,
        scratch_types=[
            pltpu.VMEM((BPT + 16,), jnp.int32),
            pltpu.VMEM((PPT,), jnp.int32),
            pltpu.VMEM((PPT,), jnp.int32),
            pltpu.VMEM((DPC, BPT), jnp.float32),
            pltpu.VMEM((PPT,), jnp.float32),
            pltpu.VMEM((PPT,), jnp.float32),
            pltpu.VMEM((2, DPC * NTAIL), jnp.float32),
            pltpu.VMEM((8, 8, 128), jnp.float32),
            pltpu.VMEM_SHARED((NWORDS,), jnp.float32),
            pltpu.SemaphoreType.DMA,
            pltpu.SemaphoreType.DMA,
            pltpu.SemaphoreType.DMA,
        ],
        compiler_params=params,
    )
    p0, p1 = f(doc_ids, tn_flat, dt_tab, o_tab, dtail, otail)
    comb = pl.kernel(
        _combine_body,
        out_type=jax.ShapeDtypeStruct((NP,), jnp.float32),
        mesh=mesh,
        scratch_types=[
            pltpu.VMEM((NP // (NC * NS),), jnp.float32),
            pltpu.VMEM((NP // (NC * NS),), jnp.float32),
            pltpu.VMEM((NP // (NC * NS),), jnp.float32),
        ],
        compiler_params=params,
    )
    return comb(p0, p1)


def kernel(doc_ids, target_noise_ids, D, O):
    dt = D.T
    dtail = dt[:, NMAIN:].reshape(-1)
    otail = O[:, NMAIN:].reshape(-1)
    out_flat = _run(doc_ids.astype(jnp.int32),
                    target_noise_ids.astype(jnp.int32).reshape(-1),
                    dt, O, dtail, otail)
    return out_flat.reshape(B, K)


# speedup vs baseline: 1.1547x; 1.1547x over previous
import jax
import jax.numpy as jnp
from jax import lax
from jax.experimental import pallas as pl
from jax.experimental.pallas import tpu as pltpu
from jax.experimental.pallas import tpu_sc as plsc

VEC = 64
NWORDS = 1000000
NMAIN = 999936
NTAIL = NWORDS - NMAIN
B = 4096
K = 21
NP = B * K

NC, NS = 2, 16
DPC = VEC // NC
NBAND = DPC // 8
PPT = NP // NS
BPT = B // NS
CP = 128
NCH = PPT // CP

_TS = [(i * (NMAIN // 128)) // NS * 128 for i in range(NS)] + [NMAIN]
_SEGS = [(_TS[i], _TS[i + 1] - _TS[i]) for i in range(NS)]


def _body(doc_hbm, tn_hbm, dt_hbm, o_hbm, dtail_hbm, otail_hbm,
          p0_hbm, p1_hbm,
          doc_v, tn_v, bmap_v, dt_all, g_v, acc_v, tails_v, blk_v,
          row_s,
          sem_st, sem_g, sem_d):
    cid = lax.axis_index("c")
    sid = lax.axis_index("s")
    dbase = cid * DPC

    pltpu.sync_copy(doc_hbm.at[pl.ds(sid * BPT, BPT)], doc_v.at[pl.ds(0, BPT)])
    pltpu.sync_copy(tn_hbm.at[pl.ds(sid * PPT, PPT)], tn_v)

    pltpu.sync_copy(dtail_hbm.at[pl.ds(dbase * NTAIL, DPC * NTAIL)],
                    tails_v.at[0])
    pltpu.sync_copy(otail_hbm.at[pl.ds(dbase * NTAIL, DPC * NTAIL)],
                    tails_v.at[1])

    iota = jnp.arange(16, dtype=jnp.int32)

    @pl.loop(0, PPT // 16)
    def init_maps(g):
        p = sid * PPT + g * 16 + iota
        bmap_v[pl.ds(g * 16, 16)] = p // K - sid * BPT
        acc_v[pl.ds(g * 16, 16)] = jnp.zeros((16,), jnp.float32)

    def fire_stage(tab_hbm, tail_idx, d):
        for i, (st, ln) in enumerate(_SEGS):
            @pl.when(sid == i)
            def _():
                pltpu.async_copy(tab_hbm.at[d, pl.ds(st, ln)],
                                 row_s.at[pl.ds(st, ln)], sem_st)

        @pl.when(sid == 0)
        def _():
            pltpu.async_copy(
                tails_v.at[tail_idx, pl.ds((d - dbase) * NTAIL, NTAIL)],
                row_s.at[pl.ds(NMAIN, NTAIL)], sem_st)

    def wait_stage():
        for i, (st, ln) in enumerate(_SEGS):
            @pl.when(sid == i)
            def _():
                pltpu.make_async_copy(o_hbm.at[0, pl.ds(st, ln)],
                                      row_s.at[pl.ds(st, ln)],
                                      sem_st).wait()

        @pl.when(sid == 0)
        def _():
            pltpu.make_async_copy(tails_v.at[0, pl.ds(0, NTAIL)],
                                  row_s.at[pl.ds(NMAIN, NTAIL)],
                                  sem_st).wait()

    fire_stage(o_hbm, 1, dbase)

    s16 = lax.rem(iota, 8)
    half = iota // 8

    def doc_step(band, gi):
        dvec = doc_v[pl.ds(gi * 8, 16)]
        rt8 = pl.multiple_of((cid * NBAND + band) * 8, 8)
        for j in range(8):
            w = dvec[j]
            w0 = pl.multiple_of(
                jnp.minimum(w // 128 * 128, NMAIN - 128), 128)
            pltpu.async_copy(dt_hbm.at[pl.ds(rt8, 8), pl.ds(w0, 128)],
                             blk_v.at[j], sem_d)
        for j2 in range(4):
            for j in (2 * j2, 2 * j2 + 1):
                pltpu.make_async_copy(
                    dt_hbm.at[pl.ds(0, 8), pl.ds(0, 128)],
                    blk_v.at[j], sem_d).wait()
            wa, wb = dvec[2 * j2], dvec[2 * j2 + 1]
            wvec = jnp.where(iota < 8, jnp.full((16,), 0, jnp.int32) + wa,
                             jnp.full((16,), 0, jnp.int32) + wb)
            lane_m = wvec - jnp.minimum(wvec // 128 * 128, NMAIN - 128)
            jt = jnp.maximum(wvec - NMAIN, 0)
            jvec = 2 * j2 + half
            vals_m = plsc.load_gather(blk_v, [jvec, s16, lane_m])
            tloc = band * 8 + s16
            vals_t = plsc.load_gather(
                tails_v, [jnp.zeros((16,), jnp.int32), tloc * NTAIL + jt])
            vals = jnp.where(wvec >= NMAIN, vals_t, vals_m)
            plsc.store_scatter(dt_all, [tloc, gi * 8 + jvec], vals)

    @pl.loop(0, BPT // 8)
    def doc_band0(gi):
        doc_step(0, gi)

    @pl.loop(0, DPC)
    def pass2(t):
        d = dbase + t
        band = t // 8 + 1

        @pl.when(band < NBAND)
        def _():
            @pl.loop(0, 4)
            def steps(q):
                doc_step(band, lax.rem(t, 8) * 4 + q)

        wait_stage()
        plsc.subcore_barrier()

        @pl.loop(0, NCH)
        def fire_g(c):
            pltpu.async_copy(row_s.at[tn_v.at[pl.ds(c * CP, CP)]],
                             g_v.at[pl.ds(c * CP, CP)], sem_g)

        pltpu.make_async_copy(o_hbm.at[0, pl.ds(0, PPT)], g_v, sem_g).wait()
        plsc.subcore_barrier()

        @pl.when(t + 1 < DPC)
        def _():
            fire_stage(o_hbm, 1, d + 1)

        tsplat = jnp.full((16,), 0, jnp.int32) + t

        @pl.loop(0, PPT // 16)
        def accum(g):
            bv = bmap_v[pl.ds(g * 16, 16)]
            dv = plsc.load_gather(dt_all, [tsplat, bv])
            plsc.addupdate(acc_v.at[pl.ds(g * 16, 16)],
                           dv * g_v[pl.ds(g * 16, 16)])

    @pl.when(cid == 0)
    def _():
        pltpu.sync_copy(acc_v, p0_hbm.at[pl.ds(sid * PPT, PPT)])

    @pl.when(cid == 1)
    def _():
        pltpu.sync_copy(acc_v, p1_hbm.at[pl.ds(sid * PPT, PPT)])


def _combine_body(p0_hbm, p1_hbm, out_hbm, v0, v1, vo):
    wid = lax.axis_index("s") * NC + lax.axis_index("c")
    n = NP // (NC * NS)
    pltpu.sync_copy(p0_hbm.at[pl.ds(wid * n, n)], v0)
    pltpu.sync_copy(p1_hbm.at[pl.ds(wid * n, n)], v1)

    @pl.loop(0, n // 16)
    def add(g):
        s = pl.ds(g * 16, 16)
        vo[s] = v0[s] + v1[s]

    pltpu.sync_copy(vo, out_hbm.at[pl.ds(wid * n, n)])


@jax.jit
def _run(doc_ids, tn_flat, dt_tab, o_tab, dtail, otail):
    mesh = plsc.VectorSubcoreMesh(core_axis_name="c", subcore_axis_name="s")
    params = pltpu.CompilerParams(use_tc_tiling_on_sc=True,
                                  needs_layout_passes=False)
    f = pl.kernel(
        _body,
        out_type=(jax.ShapeDtypeStruct((NP,), jnp.float32),
                  jax.ShapeDtypeStruct((NP,), jnp.float32)),
        mesh=mesh,
        scratch_types=[
            pltpu.VMEM((BPT + 16,), jnp.int32),
            pltpu.VMEM((PPT,), jnp.int32),
            pltpu.VMEM((PPT,), jnp.int32),
            pltpu.VMEM((DPC, BPT), jnp.float32),
            pltpu.VMEM((PPT,), jnp.float32),
            pltpu.VMEM((PPT,), jnp.float32),
            pltpu.VMEM((2, DPC * NTAIL), jnp.float32),
            pltpu.VMEM((8, 8, 128), jnp.float32),
            pltpu.VMEM_SHARED((NWORDS,), jnp.float32),
            pltpu.SemaphoreType.DMA,
            pltpu.SemaphoreType.DMA,
            pltpu.SemaphoreType.DMA,
        ],
        compiler_params=params,
    )
    p0, p1 = f(doc_ids, tn_flat, dt_tab, o_tab, dtail, otail)
    comb = pl.kernel(
        _combine_body,
        out_type=jax.ShapeDtypeStruct((NP,), jnp.float32),
        mesh=mesh,
        scratch_types=[
            pltpu.VMEM((NP // (NC * NS),), jnp.float32),
            pltpu.VMEM((NP // (NC * NS),), jnp.float32),
            pltpu.VMEM((NP // (NC * NS),), jnp.float32),
        ],
        compiler_params=params,
    )
    return comb(p0, p1)


def kernel(doc_ids, target_noise_ids, D, O):
    dt = D.T
    dtail = dt[:, NMAIN:].reshape(-1)
    otail = O[:, NMAIN:].reshape(-1)
    out_flat = _run(doc_ids.astype(jnp.int32),
                    target_noise_ids.astype(jnp.int32).reshape(-1),
                    dt, O, dtail, otail)
    return out_flat.reshape(B, K)
